# Initial kernel scaffold; baseline (speedup 1.0000x reference)
import jax
import jax.numpy as jnp
from jax.experimental import pallas as pl


def kernel(x_user, x_item, edge_index_ui, edge_index_iu, perm_user, perm_item, prompt_user, prompt_item, W_self_user, W_self_item, W_ui, W_iu, M_disc):
    raise NotImplementedError("write your pallas kernel here")


# speedup vs baseline: 3.9401x; 3.9401x over previous
import functools

import jax
import jax.numpy as jnp
from jax import lax
from jax.experimental import pallas as pl
from jax.experimental.pallas import tpu as pltpu
from jax.experimental.pallas import tpu_sc as plsc

_K = 128
_NSUB = 16
_BLK = 1000


def _sc_call(tbl, perm, esrc, edst, zeros, N, D, NPAD, NACC, CHUNKS):
    PCH_PER_TILE = NPAD // (_NSUB * _K)
    ZROWS = NACC // _NSUB
    WROWS = NACC // _NSUB
    mesh = plsc.VectorSubcoreMesh(core_axis_name="c", subcore_axis_name="s")

    @functools.partial(
        pl.kernel,
        out_type=(
            jax.ShapeDtypeStruct((2, NACC, D), jnp.float32),
            jax.ShapeDtypeStruct((2, NACC, D), jnp.float32),
            jax.ShapeDtypeStruct((2, NPAD, D), jnp.float32),
        ),
        mesh=mesh,
        scratch_types=[
            pltpu.VMEM((_K,), jnp.int32),
            pltpu.VMEM((_K,), jnp.int32),
            pltpu.VMEM((_K, D), jnp.float32),
            pltpu.VMEM_SHARED((NACC, D), jnp.float32),
            pltpu.SemaphoreType.DMA,
        ],
    )
    def sc_fn(tbl_ref, perm_ref, esrc_ref, edst_ref, zeros_ref,
              s1_ref, s2_ref, x2_ref, idx_a, idx_b, rows, acc, sem):
        c = lax.axis_index("c")
        s = lax.axis_index("s")

        pltpu.sync_copy(zeros_ref.at[pl.ds(s * ZROWS, ZROWS)],
                        acc.at[pl.ds(s * ZROWS, ZROWS)])

        def x2_body(j, carry):
            row = s * PCH_PER_TILE + j
            pltpu.sync_copy(perm_ref.at[c, row], idx_a)
            pltpu.async_copy(tbl_ref.at[c].at[idx_a], rows, sem).wait()
            pltpu.sync_copy(rows, x2_ref.at[c].at[pl.ds(row * _K, _K)])
            return carry
        lax.fori_loop(0, PCH_PER_TILE, x2_body, 0)

        plsc.subcore_barrier()

        def edge_pass(table_ref, out_ref):
            def body(j, carry):
                row = s * CHUNKS + j
                pltpu.sync_copy(esrc_ref.at[c, row], idx_a)
                pltpu.async_copy(table_ref.at[idx_a], rows, sem).wait()
                pltpu.sync_copy(edst_ref.at[c, row], idx_b)
                pltpu.sync_copy(rows, acc.at[idx_b], add=True)
                return carry
            lax.fori_loop(0, CHUNKS, body, 0)
            plsc.subcore_barrier()
            pltpu.sync_copy(acc.at[pl.ds(s * WROWS, WROWS)],
                            out_ref.at[c].at[pl.ds(s * WROWS, WROWS)])

        edge_pass(tbl_ref.at[c], s1_ref)
        plsc.subcore_barrier()
        pltpu.sync_copy(zeros_ref.at[pl.ds(s * ZROWS, ZROWS)],
                        acc.at[pl.ds(s * ZROWS, ZROWS)])
        plsc.subcore_barrier()
        edge_pass(x2_ref.at[c], s2_ref)

    return sc_fn(tbl, perm, esrc, edst, zeros)


def _tc_hidden(x_user, x_item, S1, S2, X2, p_u, p_i, Wsu, Wsi, Wiu, Wui, N, D):
    G = N // _BLK

    def body(xu, xi, s1u, s1i, s2u, s2i, x2i, x2u, pu, pi,
             wsu, wsi, wiu, wui, h1u_o, h2u_o, h1i_o, h2i_o, cs_o, acc):
        i = pl.program_id(0)
        pu_r = pu[...]
        pi_r = pi[...]

        def dot(a, w):
            return jnp.dot(a, w[...], preferred_element_type=jnp.float32)

        h1u = jnp.maximum(dot(xu[...] * pu_r, wsu) + dot(s1u[0] * pi_r, wiu), 0.0)
        h2u = jnp.maximum(dot(x2u[0] * pu_r, wsu) + dot(s2u[0] * pi_r, wiu), 0.0)
        h1i = jnp.maximum(dot(xi[...] * pi_r, wsi) + dot(s1i[0] * pu_r, wui), 0.0)
        h2i = jnp.maximum(dot(x2i[0] * pi_r, wsi) + dot(s2i[0] * pu_r, wui), 0.0)
        h1u_o[...] = h1u
        h2u_o[...] = h2u
        h1i_o[...] = h1i
        h2i_o[...] = h2i

        @pl.when(i == 0)
        def _():
            acc[...] = jnp.zeros_like(acc)
        acc[0:1, :] += jnp.sum(h1u, axis=0, keepdims=True)
        acc[1:2, :] += jnp.sum(h1i, axis=0, keepdims=True)

        @pl.when(i == G - 1)
        def _():
            cs_o[...] = acc[...]

    full = lambda i: (0, 0)
    stk = lambda k: pl.BlockSpec((1, _BLK, D), lambda i, k=k: (k, i, 0))
    blk = pl.BlockSpec((_BLK, D), lambda i: (i, 0))
    return pl.pallas_call(
        body,
        grid=(G,),
        in_specs=[
            blk, blk,
            stk(0), stk(1), stk(0), stk(1), stk(0), stk(1),
            pl.BlockSpec((1, D), full), pl.BlockSpec((1, D), full),
            pl.BlockSpec((D, D), full), pl.BlockSpec((D, D), full),
            pl.BlockSpec((D, D), full), pl.BlockSpec((D, D), full),
        ],
        out_specs=[blk, blk, blk, blk, pl.BlockSpec((2, D), full)],
        out_shape=[
            jax.ShapeDtypeStruct((N, D), jnp.float32),
            jax.ShapeDtypeStruct((N, D), jnp.float32),
            jax.ShapeDtypeStruct((N, D), jnp.float32),
            jax.ShapeDtypeStruct((N, D), jnp.float32),
            jax.ShapeDtypeStruct((2, D), jnp.float32),
        ],
        scratch_shapes=[pltpu.VMEM((2, D), jnp.float32)],
    )(x_user, x_item, S1, S1, S2, S2, X2, X2, p_u, p_i, Wsu, Wsi, Wiu, Wui)


def _tc_logits(h1u, h2u, h1i, h2i, colsums, M, N, D):
    G = N // _BLK

    def body(h1u_r, h2u_r, h1i_r, h2i_r, cs, m, o1u, o2u, o1i, o2i, gv):
        i = pl.program_id(0)

        @pl.when(i == 0)
        def _():
            cvec = jax.nn.sigmoid(cs[...] * (1.0 / N))
            mm = m[...]
            gu = jnp.sum(mm * cvec[0:1, :], axis=1, keepdims=True)
            gi = jnp.sum(mm * cvec[1:2, :], axis=1, keepdims=True)
            gv[...] = jnp.concatenate([gu, gi], axis=1)

        gu = gv[:, 0:1]
        gi = gv[:, 1:2]

        def dot(h, g):
            return jnp.dot(h[...], g, preferred_element_type=jnp.float32)

        o1u[...] = dot(h1u_r, gu)
        o2u[...] = dot(h2u_r, gu)
        o1i[...] = dot(h1i_r, gi)
        o2i[...] = dot(h2i_r, gi)

    full = lambda i: (0, 0)
    blk = pl.BlockSpec((_BLK, D), lambda i: (i, 0))
    obk = pl.BlockSpec((_BLK, 1), lambda i: (i, 0))
    oshape = jax.ShapeDtypeStruct((N, 1), jnp.float32)
    return pl.pallas_call(
        body,
        grid=(G,),
        in_specs=[blk, blk, blk, blk,
                  pl.BlockSpec((2, D), full), pl.BlockSpec((D, D), full)],
        out_specs=[obk, obk, obk, obk],
        out_shape=[oshape, oshape, oshape, oshape],
        scratch_shapes=[pltpu.VMEM((D, 2), jnp.float32)],
    )(h1u, h2u, h1i, h2i, colsums, M)


def kernel(x_user, x_item, edge_index_ui, edge_index_iu, perm_user, perm_item,
           prompt_user, prompt_item, W_self_user, W_self_item, W_ui, W_iu,
           M_disc):
    N, D = x_user.shape
    E = edge_index_ui.shape[1]
    CHUNKS = -(-E // (_NSUB * _K))
    EPAD = CHUNKS * _NSUB * _K
    PCH = -(-N // (_NSUB * _K)) * _NSUB
    NPAD = PCH * _K
    NACC = NPAD

    def pad_edges(e):
        src = jnp.concatenate([e[0], jnp.zeros((EPAD - E,), jnp.int32)])
        dst = jnp.concatenate([e[1], jnp.full((EPAD - E,), N, jnp.int32)])
        return (src.reshape(_NSUB * CHUNKS, _K),
                dst.reshape(_NSUB * CHUNKS, _K))

    src_iu, dst_iu = pad_edges(edge_index_iu)
    src_ui, dst_ui = pad_edges(edge_index_ui)
    esrc = jnp.stack([src_iu, src_ui])
    edst = jnp.stack([dst_iu, dst_ui])
    perm = jnp.stack([
        jnp.concatenate([perm_item, jnp.zeros((NPAD - N,), jnp.int32)]),
        jnp.concatenate([perm_user, jnp.zeros((NPAD - N,), jnp.int32)]),
    ]).reshape(2, PCH, _K)
    tbl = jnp.stack([x_item, x_user])
    zeros = jnp.zeros((NACC, D), jnp.float32)

    S1, S2, X2 = _sc_call(tbl, perm, esrc, edst, zeros, N, D, NPAD, NACC, CHUNKS)

    h1u, h2u, h1i, h2i, colsums = _tc_hidden(
        x_user, x_item, S1, S2, X2, prompt_user, prompt_item,
        W_self_user, W_self_item, W_iu, W_ui, N, D)

    o1u, o2u, o1i, o2i = _tc_logits(h1u, h2u, h1i, h2i, colsums, M_disc, N, D)

    return jnp.concatenate([o1u.reshape(-1), o2u.reshape(-1),
                            o1i.reshape(-1), o2i.reshape(-1)])
